# Initial kernel scaffold; baseline (speedup 1.0000x reference)
import jax
import jax.numpy as jnp
from jax.experimental import pallas as pl


def kernel(input, adj, W, a):
    raise NotImplementedError("write your pallas kernel here")


# speedup vs baseline: 1.7825x; 1.7825x over previous
import functools

import jax
import jax.numpy as jnp
from jax.experimental import pallas as pl
from jax.experimental.pallas import tpu as pltpu

ALPHA = 0.2


def _pick_block(n: int, target: int) -> int:
    b = min(target, n)
    b -= b % 8
    while b >= 8:
        if n % b == 0:
            return b
        b -= 8
    return n


def _prologue_body(x_ref, w_ref, a2_ref, wh_ref, v1_ref, v2_ref):
    wh = jnp.dot(x_ref[...], w_ref[...], preferred_element_type=jnp.float32)
    wh_ref[...] = wh
    s_dst = jnp.dot(wh, a2_ref[...], preferred_element_type=jnp.float32)
    v1_ref[...] = jnp.exp(-s_dst)
    v2_ref[...] = jnp.exp(-ALPHA * s_dst)


def _main_body(nj, bm, bn, adj_ref, wh_ref, v1_ref, v2_ref, a1_ref, out_ref,
               acc_ref, row_ref, u1_ref, u2_ref):
    i = pl.program_id(0)
    j = pl.program_id(1)

    @pl.when(j == 0)
    def _init():
        wh_i = wh_ref[pl.ds(i * bm, bm), :]
        s_src = jnp.dot(wh_i, a1_ref[...], preferred_element_type=jnp.float32)
        u1_ref[...] = jnp.exp(-s_src)
        u2_ref[...] = jnp.exp(-ALPHA * s_src)
        acc_ref[...] = jnp.zeros_like(acc_ref)
        row_ref[...] = jnp.zeros_like(row_ref)

    e = adj_ref[...] * jnp.minimum(u1_ref[...] * v1_ref[...],
                                   u2_ref[...] * v2_ref[...])
    wh_j = wh_ref[pl.ds(j * bn, bn), :]
    acc_ref[...] += jnp.dot(e, wh_j, preferred_element_type=jnp.float32)
    row_ref[...] += jnp.sum(e, axis=1, keepdims=True)

    @pl.when(j == nj - 1)
    def _finish():
        h = acc_ref[...] / row_ref[...]
        out_ref[...] = jnp.where(h > 0, h, jnp.exp(jnp.minimum(h, 0.0)) - 1.0)


def kernel(input, adj, W, a):
    n, f_in = input.shape
    f_out = W.shape[1]
    a1 = a[0, :f_out].reshape(f_out, 1)
    a2 = a[0, f_out:].reshape(f_out, 1)

    bp = _pick_block(n, 2000)
    np_ = n // bp
    wh, v1c, v2c = pl.pallas_call(
        _prologue_body,
        grid=(np_,),
        in_specs=[
            pl.BlockSpec((bp, f_in), lambda i: (i, 0)),
            pl.BlockSpec((f_in, f_out), lambda i: (0, 0)),
            pl.BlockSpec((f_out, 1), lambda i: (0, 0)),
        ],
        out_specs=[
            pl.BlockSpec((bp, f_out), lambda i: (i, 0)),
            pl.BlockSpec((bp, 1), lambda i: (i, 0)),
            pl.BlockSpec((bp, 1), lambda i: (i, 0)),
        ],
        out_shape=[
            jax.ShapeDtypeStruct((n, f_out), jnp.float32),
            jax.ShapeDtypeStruct((n, 1), jnp.float32),
            jax.ShapeDtypeStruct((n, 1), jnp.float32),
        ],
    )(input, W, a2)

    v1 = v1c.reshape(1, n)
    v2 = v2c.reshape(1, n)

    bm = _pick_block(n, 400)
    bn = n
    ni, nj = n // bm, n // bn
    out = pl.pallas_call(
        functools.partial(_main_body, nj, bm, bn),
        grid=(ni, nj),
        in_specs=[
            pl.BlockSpec((bm, bn), lambda i, j: (i, j)),
            pl.BlockSpec((n, f_out), lambda i, j: (0, 0)),
            pl.BlockSpec((1, bn), lambda i, j: (0, j)),
            pl.BlockSpec((1, bn), lambda i, j: (0, j)),
            pl.BlockSpec((f_out, 1), lambda i, j: (0, 0)),
        ],
        out_specs=pl.BlockSpec((bm, f_out), lambda i, j: (i, 0)),
        out_shape=jax.ShapeDtypeStruct((n, f_out), jnp.float32),
        scratch_shapes=[
            pltpu.VMEM((bm, f_out), jnp.float32),
            pltpu.VMEM((bm, 1), jnp.float32),
            pltpu.VMEM((bm, 1), jnp.float32),
            pltpu.VMEM((bm, 1), jnp.float32),
        ],
        compiler_params=pltpu.CompilerParams(
            dimension_semantics=("arbitrary", "arbitrary")),
    )(adj, wh, v1, v2, a1)
    return out


# speedup vs baseline: 1.8176x; 1.0197x over previous
import functools

import jax
import jax.numpy as jnp
from jax.experimental import pallas as pl
from jax.experimental.pallas import tpu as pltpu

ALPHA = 0.2


def _pick_block(n: int, target: int) -> int:
    b = min(target, n)
    b -= b % 8
    while b >= 8:
        if n % b == 0:
            return b
        b -= 8
    return n


def _prologue_body(x_ref, w_ref, a2_ref, wh_ref, v1_ref, v2_ref):
    wh = jnp.dot(x_ref[...], w_ref[...], preferred_element_type=jnp.float32)
    wh_ref[...] = wh
    s_dst = jnp.dot(wh, a2_ref[...], preferred_element_type=jnp.float32)
    v1_ref[...] = jnp.exp(-s_dst)
    v2_ref[...] = jnp.exp(-ALPHA * s_dst)


def _main_body(nj, bm, bn, adj_ref, wh_ref, v1_ref, v2_ref, a1_ref, ones_ref,
               out_ref, acc_ref, row_ref, r_ref):
    i = pl.program_id(0)
    j = pl.program_id(1)

    @pl.when(j == 0)
    def _init():
        wh_i = wh_ref[pl.ds(i * bm, bm), :]
        s_src = jnp.dot(wh_i, a1_ref[...], preferred_element_type=jnp.float32)
        r_ref[...] = jnp.exp((1.0 - ALPHA) * s_src)
        acc_ref[...] = jnp.zeros_like(acc_ref)
        row_ref[...] = jnp.zeros_like(row_ref)

    e = adj_ref[...] * jnp.minimum(v1_ref[...], r_ref[...] * v2_ref[...])
    wh_j = wh_ref[pl.ds(j * bn, bn), :]
    acc_ref[...] += jnp.dot(e, wh_j, preferred_element_type=jnp.float32)
    row_ref[...] += jnp.dot(e, ones_ref[...], preferred_element_type=jnp.float32)

    @pl.when(j == nj - 1)
    def _finish():
        h = acc_ref[...] / row_ref[...]
        out_ref[...] = jnp.where(h > 0, h, jnp.exp(jnp.minimum(h, 0.0)) - 1.0)


def kernel(input, adj, W, a):
    n, f_in = input.shape
    f_out = W.shape[1]
    a1 = a[0, :f_out].reshape(f_out, 1)
    a2 = a[0, f_out:].reshape(f_out, 1)

    bp = _pick_block(n, 2000)
    np_ = n // bp
    wh, v1c, v2c = pl.pallas_call(
        _prologue_body,
        grid=(np_,),
        in_specs=[
            pl.BlockSpec((bp, f_in), lambda i: (i, 0)),
            pl.BlockSpec((f_in, f_out), lambda i: (0, 0)),
            pl.BlockSpec((f_out, 1), lambda i: (0, 0)),
        ],
        out_specs=[
            pl.BlockSpec((bp, f_out), lambda i: (i, 0)),
            pl.BlockSpec((bp, 1), lambda i: (i, 0)),
            pl.BlockSpec((bp, 1), lambda i: (i, 0)),
        ],
        out_shape=[
            jax.ShapeDtypeStruct((n, f_out), jnp.float32),
            jax.ShapeDtypeStruct((n, 1), jnp.float32),
            jax.ShapeDtypeStruct((n, 1), jnp.float32),
        ],
    )(input, W, a2)

    v1 = v1c.reshape(1, n)
    v2 = v2c.reshape(1, n)

    bm = _pick_block(n, 400)
    bn = n
    ni, nj = n // bm, n // bn
    ones = jnp.ones((n, 1), dtype=jnp.float32)
    out = pl.pallas_call(
        functools.partial(_main_body, nj, bm, bn),
        grid=(ni, nj),
        in_specs=[
            pl.BlockSpec((bm, bn), lambda i, j: (i, j)),
            pl.BlockSpec((n, f_out), lambda i, j: (0, 0)),
            pl.BlockSpec((1, bn), lambda i, j: (0, j)),
            pl.BlockSpec((1, bn), lambda i, j: (0, j)),
            pl.BlockSpec((f_out, 1), lambda i, j: (0, 0)),
            pl.BlockSpec((bn, 1), lambda i, j: (j, 0)),
        ],
        out_specs=pl.BlockSpec((bm, f_out), lambda i, j: (i, 0)),
        out_shape=jax.ShapeDtypeStruct((n, f_out), jnp.float32),
        scratch_shapes=[
            pltpu.VMEM((bm, f_out), jnp.float32),
            pltpu.VMEM((bm, 1), jnp.float32),
            pltpu.VMEM((bm, 1), jnp.float32),
        ],
        compiler_params=pltpu.CompilerParams(
            dimension_semantics=("arbitrary", "arbitrary")),
    )(adj, wh, v1, v2, a1, ones)
    return out


# speedup vs baseline: 1.9987x; 1.0996x over previous
import functools

import jax
import jax.numpy as jnp
from jax.experimental import pallas as pl
from jax.experimental.pallas import tpu as pltpu

ALPHA = 0.2


def _pick_block(n: int, target: int) -> int:
    b = min(target, n)
    b -= b % 8
    while b >= 8:
        if n % b == 0:
            return b
        b -= 8
    return n


def _prologue_body(x_ref, w_ref, a2_ref, wh_ref, v1_ref, v2_ref):
    wh = jnp.dot(x_ref[...], w_ref[...], preferred_element_type=jnp.float32)
    f_out = wh.shape[1]
    lane = jax.lax.broadcasted_iota(jnp.int32, (wh.shape[0], 8), 1)
    wh_ref[:, :f_out] = wh
    wh_ref[:, f_out:] = jnp.where(lane == 0, 1.0, 0.0)
    s_dst = jnp.dot(wh, a2_ref[...], preferred_element_type=jnp.float32)
    v1_ref[...] = jnp.exp(-s_dst)
    v2_ref[...] = jnp.exp(-ALPHA * s_dst)


def _main_body(nj, bm, bn, f_out, adj_ref, wh_ref, v1_ref, v2_ref, a1_ref,
               out_ref, acc_ref, r_ref):
    i = pl.program_id(0)
    j = pl.program_id(1)

    @pl.when(j == 0)
    def _init():
        wh_i = wh_ref[pl.ds(i * bm, bm), :f_out]
        s_src = jnp.dot(wh_i, a1_ref[...], preferred_element_type=jnp.float32)
        r_ref[...] = jnp.exp((1.0 - ALPHA) * s_src)
        acc_ref[...] = jnp.zeros_like(acc_ref)

    e = adj_ref[...] * jnp.minimum(v1_ref[...], r_ref[...] * v2_ref[...])
    wh_j = wh_ref[pl.ds(j * bn, bn), :]
    acc_ref[...] += jnp.dot(e, wh_j, preferred_element_type=jnp.float32)

    @pl.when(j == nj - 1)
    def _finish():
        h = acc_ref[:, :f_out] / acc_ref[:, f_out:f_out + 1]
        out_ref[...] = jnp.where(h > 0, h, jnp.exp(jnp.minimum(h, 0.0)) - 1.0)


def kernel(input, adj, W, a):
    n, f_in = input.shape
    f_out = W.shape[1]
    a1 = a[0, :f_out].reshape(f_out, 1)
    a2 = a[0, f_out:].reshape(f_out, 1)

    bp = _pick_block(n, 2000)
    np_ = n // bp
    wh, v1c, v2c = pl.pallas_call(
        _prologue_body,
        grid=(np_,),
        in_specs=[
            pl.BlockSpec((bp, f_in), lambda i: (i, 0)),
            pl.BlockSpec((f_in, f_out), lambda i: (0, 0)),
            pl.BlockSpec((f_out, 1), lambda i: (0, 0)),
        ],
        out_specs=[
            pl.BlockSpec((bp, f_out + 8), lambda i: (i, 0)),
            pl.BlockSpec((bp, 1), lambda i: (i, 0)),
            pl.BlockSpec((bp, 1), lambda i: (i, 0)),
        ],
        out_shape=[
            jax.ShapeDtypeStruct((n, f_out + 8), jnp.float32),
            jax.ShapeDtypeStruct((n, 1), jnp.float32),
            jax.ShapeDtypeStruct((n, 1), jnp.float32),
        ],
    )(input, W, a2)

    v1 = v1c.reshape(1, n)
    v2 = v2c.reshape(1, n)

    bm = _pick_block(n, 400)
    bn = n
    ni, nj = n // bm, n // bn
    out = pl.pallas_call(
        functools.partial(_main_body, nj, bm, bn, f_out),
        grid=(ni, nj),
        in_specs=[
            pl.BlockSpec((bm, bn), lambda i, j: (i, j)),
            pl.BlockSpec((n, f_out + 8), lambda i, j: (0, 0)),
            pl.BlockSpec((1, bn), lambda i, j: (0, j)),
            pl.BlockSpec((1, bn), lambda i, j: (0, j)),
            pl.BlockSpec((f_out, 1), lambda i, j: (0, 0)),
        ],
        out_specs=pl.BlockSpec((bm, f_out), lambda i, j: (i, 0)),
        out_shape=jax.ShapeDtypeStruct((n, f_out), jnp.float32),
        scratch_shapes=[
            pltpu.VMEM((bm, f_out + 8), jnp.float32),
            pltpu.VMEM((bm, 1), jnp.float32),
        ],
        compiler_params=pltpu.CompilerParams(
            dimension_semantics=("arbitrary", "arbitrary")),
    )(adj, wh, v1, v2, a1)
    return out


# speedup vs baseline: 2.0319x; 1.0166x over previous
import functools

import jax
import jax.numpy as jnp
from jax.experimental import pallas as pl
from jax.experimental.pallas import tpu as pltpu

ALPHA = 0.2


def _pick_block(n: int, target: int) -> int:
    b = min(target, n)
    b -= b % 8
    while b >= 8:
        if n % b == 0:
            return b
        b -= 8
    return n


def _prologue_body(x_ref, w_ref, a1_ref, a2_ref, wh_ref, c_ref, r_ref):
    wh = jnp.dot(x_ref[...], w_ref[...], preferred_element_type=jnp.float32)
    f_out = wh.shape[1]
    s_dst = jnp.dot(wh, a2_ref[...], preferred_element_type=jnp.float32)
    s_src = jnp.dot(wh, a1_ref[...], preferred_element_type=jnp.float32)
    v2 = jnp.exp(-ALPHA * s_dst)
    c_ref[...] = jnp.exp(-(1.0 - ALPHA) * s_dst)
    r_ref[...] = jnp.exp((1.0 - ALPHA) * s_src)
    lane = jax.lax.broadcasted_iota(jnp.int32, (wh.shape[0], 8), 1)
    wh_ref[:, :f_out] = v2 * wh
    wh_ref[:, f_out:] = jnp.where(lane == 0, v2, 0.0)


def _main_body(nj, bm, bn, f_out, adj_ref, wh_ref, c_ref, r_ref,
               out_ref, acc_ref):
    j = pl.program_id(1)

    @pl.when(j == 0)
    def _init():
        acc_ref[...] = jnp.zeros_like(acc_ref)

    e = adj_ref[...] * jnp.minimum(c_ref[...], r_ref[...])
    wh_j = wh_ref[pl.ds(j * bn, bn), :]
    acc_ref[...] += jnp.dot(e, wh_j, preferred_element_type=jnp.float32)

    @pl.when(j == nj - 1)
    def _finish():
        h = acc_ref[:, :f_out] / acc_ref[:, f_out:f_out + 1]
        out_ref[...] = jnp.where(h > 0, h, jnp.exp(jnp.minimum(h, 0.0)) - 1.0)


def kernel(input, adj, W, a):
    n, f_in = input.shape
    f_out = W.shape[1]
    a1 = a[0, :f_out].reshape(f_out, 1)
    a2 = a[0, f_out:].reshape(f_out, 1)

    bp = _pick_block(n, 2000)
    np_ = n // bp
    wh, cc, rc = pl.pallas_call(
        _prologue_body,
        grid=(np_,),
        in_specs=[
            pl.BlockSpec((bp, f_in), lambda i: (i, 0)),
            pl.BlockSpec((f_in, f_out), lambda i: (0, 0)),
            pl.BlockSpec((f_out, 1), lambda i: (0, 0)),
            pl.BlockSpec((f_out, 1), lambda i: (0, 0)),
        ],
        out_specs=[
            pl.BlockSpec((bp, f_out + 8), lambda i: (i, 0)),
            pl.BlockSpec((bp, 1), lambda i: (i, 0)),
            pl.BlockSpec((bp, 1), lambda i: (i, 0)),
        ],
        out_shape=[
            jax.ShapeDtypeStruct((n, f_out + 8), jnp.float32),
            jax.ShapeDtypeStruct((n, 1), jnp.float32),
            jax.ShapeDtypeStruct((n, 1), jnp.float32),
        ],
    )(input, W, a1, a2)

    c = cc.reshape(1, n)

    bm = _pick_block(n, 400)
    bn = n
    ni, nj = n // bm, n // bn
    out = pl.pallas_call(
        functools.partial(_main_body, nj, bm, bn, f_out),
        grid=(ni, nj),
        in_specs=[
            pl.BlockSpec((bm, bn), lambda i, j: (i, j)),
            pl.BlockSpec((n, f_out + 8), lambda i, j: (0, 0)),
            pl.BlockSpec((1, bn), lambda i, j: (0, j)),
            pl.BlockSpec((bm, 1), lambda i, j: (i, 0)),
        ],
        out_specs=pl.BlockSpec((bm, f_out), lambda i, j: (i, 0)),
        out_shape=jax.ShapeDtypeStruct((n, f_out), jnp.float32),
        scratch_shapes=[
            pltpu.VMEM((bm, f_out + 8), jnp.float32),
        ],
        compiler_params=pltpu.CompilerParams(
            dimension_semantics=("arbitrary", "arbitrary")),
    )(adj, wh, c, rc)
    return out


# speedup vs baseline: 2.0328x; 1.0004x over previous
import functools

import jax
import jax.numpy as jnp
from jax.experimental import pallas as pl
from jax.experimental.pallas import tpu as pltpu

ALPHA = 0.2


def _pick_block(n: int, target: int) -> int:
    b = min(target, n)
    b -= b % 8
    while b >= 8:
        if n % b == 0:
            return b
        b -= 8
    return n


def _prologue_body(x_ref, w_ref, a1_ref, a2_ref, wh_ref, c_ref, r_ref):
    wh = jnp.dot(x_ref[...], w_ref[...], preferred_element_type=jnp.float32)
    f_out = wh.shape[1]
    s_dst = jnp.dot(wh, a2_ref[...], preferred_element_type=jnp.float32)
    s_src = jnp.dot(wh, a1_ref[...], preferred_element_type=jnp.float32)
    v2 = jnp.exp(-ALPHA * s_dst)
    c_ref[...] = jnp.exp(-(1.0 - ALPHA) * s_dst)
    r_ref[...] = jnp.exp((1.0 - ALPHA) * s_src)
    lane = jax.lax.broadcasted_iota(jnp.int32, (wh.shape[0], 8), 1)
    wh_ref[:, :f_out] = v2 * wh
    wh_ref[:, f_out:] = jnp.where(lane == 0, v2, 0.0)


def _main_body(nj, bm, bn, f_out, adj_ref, wh_ref, c_ref, r_ref,
               out_ref, acc_ref):
    j = pl.program_id(1)

    @pl.when(j == 0)
    def _init():
        acc_ref[...] = jnp.zeros_like(acc_ref)

    e = adj_ref[...] * jnp.minimum(c_ref[...], r_ref[...])
    wh_j = wh_ref[pl.ds(j * bn, bn), :]
    acc_ref[...] += jnp.dot(e, wh_j, preferred_element_type=jnp.float32)

    @pl.when(j == nj - 1)
    def _finish():
        h = acc_ref[:, :f_out] / acc_ref[:, f_out:f_out + 1]
        out_ref[...] = jnp.where(h > 0, h, jnp.exp(jnp.minimum(h, 0.0)) - 1.0)


def kernel(input, adj, W, a):
    n, f_in = input.shape
    f_out = W.shape[1]
    a1 = a[0, :f_out].reshape(f_out, 1)
    a2 = a[0, f_out:].reshape(f_out, 1)

    bp = _pick_block(n, 2000)
    np_ = n // bp
    wh, cc, rc = pl.pallas_call(
        _prologue_body,
        grid=(np_,),
        in_specs=[
            pl.BlockSpec((bp, f_in), lambda i: (i, 0)),
            pl.BlockSpec((f_in, f_out), lambda i: (0, 0)),
            pl.BlockSpec((f_out, 1), lambda i: (0, 0)),
            pl.BlockSpec((f_out, 1), lambda i: (0, 0)),
        ],
        out_specs=[
            pl.BlockSpec((bp, f_out + 8), lambda i: (i, 0)),
            pl.BlockSpec((bp, 1), lambda i: (i, 0)),
            pl.BlockSpec((bp, 1), lambda i: (i, 0)),
        ],
        out_shape=[
            jax.ShapeDtypeStruct((n, f_out + 8), jnp.float32),
            jax.ShapeDtypeStruct((n, 1), jnp.float32),
            jax.ShapeDtypeStruct((n, 1), jnp.float32),
        ],
    )(input, W, a1, a2)

    c = cc.reshape(1, n)

    bm = _pick_block(n, 400)
    bn = n
    ni, nj = n // bm, n // bn
    out = pl.pallas_call(
        functools.partial(_main_body, nj, bm, bn, f_out),
        grid=(ni, nj),
        in_specs=[
            pl.BlockSpec((bm, bn), lambda i, j: (i, j)),
            pl.BlockSpec((n, f_out + 8), lambda i, j: (0, 0)),
            pl.BlockSpec((1, bn), lambda i, j: (0, j)),
            pl.BlockSpec((bm, 1), lambda i, j: (i, 0)),
        ],
        out_specs=pl.BlockSpec((bm, f_out), lambda i, j: (i, 0)),
        out_shape=jax.ShapeDtypeStruct((n, f_out), jnp.float32),
        scratch_shapes=[
            pltpu.VMEM((bm, f_out + 8), jnp.float32),
        ],
        compiler_params=pltpu.CompilerParams(
            dimension_semantics=("parallel", "arbitrary")),
    )(adj, wh, c, rc)
    return out
